# baseline (speedup 1.0000x reference)
import functools

import jax
import jax.numpy as jnp
from jax import lax
from jax.experimental import pallas as pl
from jax.experimental.pallas import tpu as pltpu
from jax.experimental.pallas import tpu_sc as plsc

_N = 52
_NP = 64
_E = 832
_WIN = 5
_HID = 512
_OUT = 128
_LANES = 16
_EG = _E // _LANES


def _sc_edge_scatter_body(edge_hbm, ew_hbm, out_hbm, acc_v, edge_v, ew_v,
                          sem1, sem2):
    cid = lax.axis_index("c")
    sid = lax.axis_index("s")

    @pl.when(sid == 0)
    def _():
        cp_edge = pltpu.make_async_copy(edge_hbm, edge_v, sem1)
        cp_ew = pltpu.make_async_copy(ew_hbm, ew_v, sem2)
        cp_edge.start()
        cp_ew.start()
        zero = jnp.zeros((_LANES,), jnp.float32)

        def zbody(r, carry):
            for c in range(_NP // _LANES):
                acc_v[r, pl.ds(pl.multiple_of(c * _LANES, _LANES), _LANES)] \
                    = zero
            return carry

        lax.fori_loop(0, _N, zbody, 0)
        cp_edge.wait()
        cp_ew.wait()
        base = cid * (_EG // 2)

        def sbody(g, carry):
            off = pl.multiple_of((base + g) * _LANES, _LANES)
            s = edge_v[0, pl.ds(off, _LANES)]
            d = edge_v[1, pl.ds(off, _LANES)]
            w = ew_v[pl.ds(off, _LANES)]
            plsc.addupdate_scatter(acc_v, [d, s], w)
            return carry

        lax.fori_loop(0, _EG // 2, sbody, 0)
        pltpu.sync_copy(acc_v, out_hbm.at[cid])


@functools.cache
def _sc_edge_scatter():
    return pl.kernel(
        _sc_edge_scatter_body,
        out_type=jax.ShapeDtypeStruct((2, _N, _NP), jnp.float32),
        mesh=plsc.VectorSubcoreMesh(core_axis_name="c", subcore_axis_name="s"),
        compiler_params=pltpu.CompilerParams(needs_layout_passes=False),
        scratch_types=[
            pltpu.VMEM((_N, _NP), jnp.float32),
            pltpu.VMEM((2, _E), jnp.int32),
            pltpu.VMEM((_E,), jnp.float32),
            pltpu.SemaphoreType.DMA,
            pltpu.SemaphoreType.DMA,
        ],
    )


def _tc_temporal_body(x_ref, w_ref, b_ref, aw_ref, h_ref, wb0, wb1, s0, s1):
    bufs, sems = (wb0, wb1), (s0, s1)
    cps = [pltpu.make_async_copy(w_ref.at[t], bufs[t % 2], sems[t % 2])
           for t in range(_WIN)]
    cps[0].start()
    cps[1].start()
    hs = []
    for t in range(_WIN):
        cps[t].wait()
        hs.append(jnp.dot(x_ref[t], bufs[t % 2][...],
                          preferred_element_type=jnp.float32))
        if t + 2 < _WIN:
            cps[t + 2].start()
    att = aw_ref[...]
    ss = [jnp.sum(h * att, axis=1, keepdims=True) for h in hs]
    m = ss[0]
    for s in ss[1:]:
        m = jnp.maximum(m, s)
    es = [jnp.exp(s - m) for s in ss]
    z = es[0]
    for e in es[1:]:
        z = z + e
    h = es[0] * hs[0]
    for t in range(1, _WIN):
        h = h + es[t] * hs[t]
    h_ref[...] = h / z + b_ref[...]


def _tc_gcn_body(h_ref, W1_ref, b1_ref, W2_ref, b2_ref,
                 g1_ref, be1_ref, g2_ref, be2_ref, lw_ref, lb_ref, acc_ref,
                 o_ref):
    a_raw = acc_ref[0, :, :_N] + acc_ref[1, :, :_N]
    deg = jnp.sum(a_raw, axis=1, keepdims=True) + 1.0
    dinv = lax.rsqrt(deg)
    rr = lax.broadcasted_iota(jnp.int32, (_N, _N), 0)
    cc = lax.broadcasted_iota(jnp.int32, (_N, _N), 1)
    a_n = jnp.where(rr == cc, a_raw + 1.0, a_raw)

    bn_rows = jnp.concatenate(
        [g1_ref[...], be1_ref[...], g2_ref[...], be2_ref[...]], axis=0)
    bn = jnp.transpose(bn_rows)

    def gcn(hin, W_r, bb_r):
        hw = jnp.dot(hin, W_r[...], preferred_element_type=jnp.float32)
        agg = jnp.dot(a_n, dinv * hw, preferred_element_type=jnp.float32)
        return dinv * agg + bb_r[...]

    def norm_relu(v, g, be):
        mean = jnp.mean(v, axis=1, keepdims=True)
        cen = v - mean
        var = jnp.mean(cen * cen, axis=1, keepdims=True)
        vn = cen * lax.rsqrt(var + 1e-5) * g + be
        return jnp.maximum(vn, 0.0)

    h1 = norm_relu(gcn(h_ref[...], W1_ref, b1_ref), bn[:, 0:1], bn[:, 1:2])
    h2 = norm_relu(gcn(h1, W2_ref, b2_ref), bn[:, 2:3], bn[:, 3:4])
    o_ref[...] = (jnp.dot(h2, lw_ref[...], preferred_element_type=jnp.float32)
                  + lb_ref[...])


def _tc_temporal_call(args, interpret=False):
    in_specs = [
        pl.BlockSpec(memory_space=pltpu.VMEM),
        pl.BlockSpec(memory_space=pl.ANY),
        pl.BlockSpec(memory_space=pltpu.VMEM),
        pl.BlockSpec(memory_space=pltpu.VMEM),
    ]
    return pl.pallas_call(
        _tc_temporal_body,
        in_specs=in_specs,
        out_shape=jax.ShapeDtypeStruct((_N, _HID), jnp.float32),
        scratch_shapes=[pltpu.VMEM((512, _HID), jnp.float32),
                        pltpu.VMEM((512, _HID), jnp.float32),
                        pltpu.SemaphoreType.DMA,
                        pltpu.SemaphoreType.DMA],
        interpret=interpret,
    )(*args)


def _tc_gcn_call(args, interpret=False):
    return pl.pallas_call(
        _tc_gcn_body,
        out_shape=jax.ShapeDtypeStruct((_N, _OUT), jnp.float32),
        interpret=interpret,
    )(*args)


def kernel(x, edge_index, edge_weight, weight, bias, attn_w, W1, b1, W2, b2,
           bn1_g, bn1_b, bn2_g, bn2_b, lin_W, lin_b):
    acc = _sc_edge_scatter()(jnp.asarray(edge_index, jnp.int32),
                             jnp.asarray(edge_weight, jnp.float32))
    h = _tc_temporal_call((x, weight, bias.reshape(1, -1),
                           attn_w.reshape(1, -1)))
    return _tc_gcn_call((
        h,
        W1, b1.reshape(1, -1), W2, b2.reshape(1, -1),
        bn1_g.reshape(1, -1), bn1_b.reshape(1, -1),
        bn2_g.reshape(1, -1), bn2_b.reshape(1, -1),
        lin_W, lin_b.reshape(1, -1), acc,
    ))


# speedup vs baseline: 1.0372x; 1.0372x over previous
import functools

import jax
import jax.numpy as jnp
from jax import lax
from jax.experimental import pallas as pl
from jax.experimental.pallas import tpu as pltpu
from jax.experimental.pallas import tpu_sc as plsc

_N = 52
_NP = 64
_E = 832
_WIN = 5
_HID = 512
_OUT = 128
_LANES = 16
_EG = _E // _LANES


def _sc_edge_scatter_body(edge_hbm, ew_hbm, out_hbm, acc_v, edge_v, ew_v,
                          sem1, sem2):
    cid = lax.axis_index("c")
    sid = lax.axis_index("s")

    @pl.when(sid == 0)
    def _():
        cp_edge = pltpu.make_async_copy(edge_hbm, edge_v, sem1)
        cp_ew = pltpu.make_async_copy(ew_hbm, ew_v, sem2)
        cp_edge.start()
        cp_ew.start()
        zero = jnp.zeros((_LANES,), jnp.float32)

        def zbody(r, carry):
            for c in range(_NP // _LANES):
                acc_v[r, pl.ds(pl.multiple_of(c * _LANES, _LANES), _LANES)] \
                    = zero
            return carry

        lax.fori_loop(0, _N, zbody, 0)
        cp_edge.wait()
        cp_ew.wait()
        base = cid * (_EG // 2)

        def sbody(g, carry):
            off = pl.multiple_of((base + g) * _LANES, _LANES)
            s = edge_v[0, pl.ds(off, _LANES)]
            d = edge_v[1, pl.ds(off, _LANES)]
            w = ew_v[pl.ds(off, _LANES)]
            plsc.addupdate_scatter(acc_v, [d, s], w)
            return carry

        lax.fori_loop(0, _EG // 2, sbody, 0)
        pltpu.sync_copy(acc_v, out_hbm.at[cid])


@functools.cache
def _sc_edge_scatter():
    return pl.kernel(
        _sc_edge_scatter_body,
        out_type=jax.ShapeDtypeStruct((2, _N, _NP), jnp.float32),
        mesh=plsc.VectorSubcoreMesh(core_axis_name="c", subcore_axis_name="s"),
        compiler_params=pltpu.CompilerParams(needs_layout_passes=False),
        scratch_types=[
            pltpu.VMEM((_N, _NP), jnp.float32),
            pltpu.VMEM((2, _E), jnp.int32),
            pltpu.VMEM((_E,), jnp.float32),
            pltpu.SemaphoreType.DMA,
            pltpu.SemaphoreType.DMA,
        ],
    )


def _tc_temporal_body(x_ref, w_ref, b_ref, aw_ref, h_ref):
    hs = [jnp.dot(x_ref[t], w_ref[t], preferred_element_type=jnp.float32)
          for t in range(_WIN)]
    att = aw_ref[...]
    ss = [jnp.sum(h * att, axis=1, keepdims=True) for h in hs]
    m = ss[0]
    for s in ss[1:]:
        m = jnp.maximum(m, s)
    es = [jnp.exp(s - m) for s in ss]
    z = es[0]
    for e in es[1:]:
        z = z + e
    h = es[0] * hs[0]
    for t in range(1, _WIN):
        h = h + es[t] * hs[t]
    h_ref[...] = h / z + b_ref[...]


def _tc_gcn_body(h_ref, W1_ref, b1_ref, W2_ref, b2_ref,
                 g1_ref, be1_ref, g2_ref, be2_ref, lw_ref, lb_ref, acc_ref,
                 o_ref):
    a_raw = acc_ref[0, :, :_N] + acc_ref[1, :, :_N]
    deg = jnp.sum(a_raw, axis=1, keepdims=True) + 1.0
    dinv = lax.rsqrt(deg)
    rr = lax.broadcasted_iota(jnp.int32, (_N, _N), 0)
    cc = lax.broadcasted_iota(jnp.int32, (_N, _N), 1)
    a_n = jnp.where(rr == cc, a_raw + 1.0, a_raw)

    bn_rows = jnp.concatenate(
        [g1_ref[...], be1_ref[...], g2_ref[...], be2_ref[...]], axis=0)
    bn = jnp.transpose(bn_rows)

    def gcn(hin, W_r, bb_r):
        hw = jnp.dot(hin, W_r[...], preferred_element_type=jnp.float32)
        agg = jnp.dot(a_n, dinv * hw, preferred_element_type=jnp.float32)
        return dinv * agg + bb_r[...]

    def norm_relu(v, g, be):
        mean = jnp.mean(v, axis=1, keepdims=True)
        cen = v - mean
        var = jnp.mean(cen * cen, axis=1, keepdims=True)
        vn = cen * lax.rsqrt(var + 1e-5) * g + be
        return jnp.maximum(vn, 0.0)

    h1 = norm_relu(gcn(h_ref[...], W1_ref, b1_ref), bn[:, 0:1], bn[:, 1:2])
    h2 = norm_relu(gcn(h1, W2_ref, b2_ref), bn[:, 2:3], bn[:, 3:4])
    o_ref[...] = (jnp.dot(h2, lw_ref[...], preferred_element_type=jnp.float32)
                  + lb_ref[...])


def _tc_temporal_call(args, interpret=False):
    return pl.pallas_call(
        _tc_temporal_body,
        out_shape=jax.ShapeDtypeStruct((_N, _HID), jnp.float32),
        interpret=interpret,
    )(*args)


def _tc_gcn_call(args, interpret=False):
    return pl.pallas_call(
        _tc_gcn_body,
        out_shape=jax.ShapeDtypeStruct((_N, _OUT), jnp.float32),
        interpret=interpret,
    )(*args)


def kernel(x, edge_index, edge_weight, weight, bias, attn_w, W1, b1, W2, b2,
           bn1_g, bn1_b, bn2_g, bn2_b, lin_W, lin_b):
    acc = _sc_edge_scatter()(jnp.asarray(edge_index, jnp.int32),
                             jnp.asarray(edge_weight, jnp.float32))
    h = _tc_temporal_call((x, weight, bias.reshape(1, -1),
                           attn_w.reshape(1, -1)))
    return _tc_gcn_call((
        h,
        W1, b1.reshape(1, -1), W2, b2.reshape(1, -1),
        bn1_g.reshape(1, -1), bn1_b.reshape(1, -1),
        bn2_g.reshape(1, -1), bn2_b.reshape(1, -1),
        lin_W, lin_b.reshape(1, -1), acc,
    ))
